# baseline (device time: 326221 ns/iter reference)
import jax
import jax.numpy as jnp
from jax import lax
from jax.experimental import pallas as pl
from jax.experimental.pallas import tpu as pltpu

N_DEV = 32
N_TOK = 2048
D_IN = 512
D_OUT = 1024
E_LOCAL = 4
CAP = 12
CHUNK = N_TOK // N_DEV


def kernel(x, router_W, route_idx, expert_W):
    del router_W

    def body(x_ref, idx_ref, w_ref, out_ref, accum_ref, stage_ref,
             send_sem, rs_recv_sems, ag_recv_sems):
        my = lax.axis_index("i")
        left = lax.rem(my - 1 + N_DEV, N_DEV)
        right = lax.rem(my + 1, N_DEV)

        barrier_sem = pltpu.get_barrier_semaphore()
        pl.semaphore_signal(barrier_sem, inc=1, device_id=(left,),
                            device_id_type=pl.DeviceIdType.MESH)
        pl.semaphore_signal(barrier_sem, inc=1, device_id=(right,),
                            device_id_type=pl.DeviceIdType.MESH)
        pl.semaphore_wait(barrier_sem, 2)

        route = idx_ref[:, :]
        base = my * E_LOCAL
        e_ids = base + lax.broadcasted_iota(jnp.int32, (1, E_LOCAL), 1)
        oh = (route == e_ids).astype(jnp.float32)
        row = lax.broadcasted_iota(jnp.int32, (N_TOK, N_TOK), 0)
        col = lax.broadcasted_iota(jnp.int32, (N_TOK, N_TOK), 1)
        tri = (col <= row).astype(jnp.float32)
        rank_incl = jnp.dot(tri, oh, preferred_element_type=jnp.float32)
        mask = oh * (rank_incl <= CAP).astype(jnp.float32)

        xv = x_ref[:, :]
        accum_ref[:, :] = jnp.dot(xv * mask[:, 0:1], w_ref[0],
                                  preferred_element_type=jnp.float32)
        for l in range(1, E_LOCAL):
            accum_ref[:, :] += jnp.dot(xv * mask[:, l:l + 1], w_ref[l],
                                       preferred_element_type=jnp.float32)

        for t in range(N_DEV - 1):
            send_c = lax.rem(my - t + N_DEV, N_DEV)
            rdma = pltpu.make_async_remote_copy(
                src_ref=accum_ref.at[pl.ds(send_c * CHUNK, CHUNK), :],
                dst_ref=stage_ref.at[t],
                send_sem=send_sem,
                recv_sem=rs_recv_sems.at[t],
                device_id=(right,),
                device_id_type=pl.DeviceIdType.MESH,
            )
            rdma.start()
            rdma.wait()
            recv_c = lax.rem(my - t - 1 + N_DEV, N_DEV)
            off = recv_c * CHUNK
            accum_ref[pl.ds(off, CHUNK), :] = (
                accum_ref[pl.ds(off, CHUNK), :] + stage_ref[t]
            )

        red_c = lax.rem(my + 1, N_DEV)
        red_off = red_c * CHUNK
        out_ref[pl.ds(red_off, CHUNK), :] = accum_ref[pl.ds(red_off, CHUNK), :]
        for t in range(N_DEV - 1):
            send_c = lax.rem(my + 1 - t + N_DEV, N_DEV)
            off = send_c * CHUNK
            rdma = pltpu.make_async_remote_copy(
                src_ref=out_ref.at[pl.ds(off, CHUNK), :],
                dst_ref=out_ref.at[pl.ds(off, CHUNK), :],
                send_sem=send_sem,
                recv_sem=ag_recv_sems.at[t],
                device_id=(right,),
                device_id_type=pl.DeviceIdType.MESH,
            )
            rdma.start()
            rdma.wait()

    return pl.pallas_call(
        body,
        out_shape=jax.ShapeDtypeStruct((N_TOK, D_OUT), jnp.float32),
        in_specs=[
            pl.BlockSpec(memory_space=pltpu.VMEM),
            pl.BlockSpec(memory_space=pltpu.VMEM),
            pl.BlockSpec(memory_space=pltpu.VMEM),
        ],
        out_specs=pl.BlockSpec(memory_space=pltpu.VMEM),
        scratch_shapes=[
            pltpu.VMEM((N_TOK, D_OUT), jnp.float32),
            pltpu.VMEM((N_DEV - 1, CHUNK, D_OUT), jnp.float32),
            pltpu.SemaphoreType.DMA,
            pltpu.SemaphoreType.DMA((N_DEV - 1,)),
            pltpu.SemaphoreType.DMA((N_DEV - 1,)),
        ],
        compiler_params=pltpu.CompilerParams(
            collective_id=0,
            vmem_limit_bytes=100 * 1024 * 1024,
        ),
    )(x, route_idx, expert_W)


# device time: 186448 ns/iter; 1.7497x vs baseline; 1.7497x over previous
import jax
import jax.numpy as jnp
from jax import lax
from jax.experimental import pallas as pl
from jax.experimental.pallas import tpu as pltpu

N_DEV = 32
N_TOK = 2048
D_IN = 512
D_OUT = 1024
N_EXP = 128
E_LOCAL = 4
CAP = 12


def kernel(x, router_W, route_idx, expert_W):
    del router_W

    def body(x_ref, idx_ref, w_ref, out_ref, g_ref, send_sem, ag_sems):
        my = lax.axis_index("i")
        left = lax.rem(my - 1 + N_DEV, N_DEV)
        right = lax.rem(my + 1, N_DEV)

        barrier_sem = pltpu.get_barrier_semaphore()
        pl.semaphore_signal(barrier_sem, inc=1, device_id=(left,),
                            device_id_type=pl.DeviceIdType.MESH)
        pl.semaphore_signal(barrier_sem, inc=1, device_id=(right,),
                            device_id_type=pl.DeviceIdType.MESH)
        pl.semaphore_wait(barrier_sem, 2)

        route = idx_ref[:, :]
        eall = lax.broadcasted_iota(jnp.int32, (1, N_EXP), 1)
        oh_all = (route == eall).astype(jnp.float32)
        row = lax.broadcasted_iota(jnp.int32, (N_TOK, N_TOK), 0)
        col = lax.broadcasted_iota(jnp.int32, (N_TOK, N_TOK), 1)
        tri = (col <= row).astype(jnp.float32)
        rank_all = jnp.dot(tri, oh_all, preferred_element_type=jnp.float32)

        base = my * E_LOCAL
        e_loc = base + lax.broadcasted_iota(jnp.int32, (1, E_LOCAL), 1)
        oh_loc = (route == e_loc).astype(jnp.float32)
        rank_loc = jnp.dot(tri, oh_loc, preferred_element_type=jnp.float32)

        xv = x_ref[:, :]
        rr = (lax.broadcasted_iota(jnp.int32, (1, CAP), 1) + 1).astype(
            jnp.float32)
        for l in range(E_LOCAL):
            sel = oh_loc[:, l:l + 1] * (rank_loc[:, l:l + 1] == rr).astype(
                jnp.float32)
            xc = lax.dot_general(
                sel, xv, dimension_numbers=(((0,), (0,)), ((), ())),
                preferred_element_type=jnp.float32)
            yc = jnp.dot(xc, w_ref[l], preferred_element_type=jnp.float32)
            g_ref[pl.ds(my, 1), l, :, :] = yc.reshape(1, CAP, D_OUT)

        for t in range(N_DEV - 1):
            bs = lax.rem(my - t + N_DEV, N_DEV)
            rdma = pltpu.make_async_remote_copy(
                src_ref=g_ref.at[pl.ds(bs, 1)],
                dst_ref=g_ref.at[pl.ds(bs, 1)],
                send_sem=send_sem,
                recv_sem=ag_sems.at[t],
                device_id=(right,),
                device_id_type=pl.DeviceIdType.MESH,
            )
            rdma.start()
            rdma.wait()

        for r in range(CAP):
            p_r = oh_all * (rank_all == float(r + 1)).astype(jnp.float32)
            g_r = g_ref[:, :, r, :].reshape(N_EXP, D_OUT)
            contrib = jnp.dot(p_r, g_r, preferred_element_type=jnp.float32)
            if r == 0:
                out_ref[:, :] = contrib
            else:
                out_ref[:, :] += contrib

    return pl.pallas_call(
        body,
        out_shape=jax.ShapeDtypeStruct((N_TOK, D_OUT), jnp.float32),
        in_specs=[
            pl.BlockSpec(memory_space=pltpu.VMEM),
            pl.BlockSpec(memory_space=pltpu.VMEM),
            pl.BlockSpec(memory_space=pltpu.VMEM),
        ],
        out_specs=pl.BlockSpec(memory_space=pltpu.VMEM),
        scratch_shapes=[
            pltpu.VMEM((N_DEV, E_LOCAL, CAP, D_OUT), jnp.float32),
            pltpu.SemaphoreType.DMA,
            pltpu.SemaphoreType.DMA((N_DEV - 1,)),
        ],
        compiler_params=pltpu.CompilerParams(
            collective_id=0,
            vmem_limit_bytes=100 * 1024 * 1024,
        ),
    )(x, route_idx, expert_W)


# device time: 149846 ns/iter; 2.1770x vs baseline; 1.2443x over previous
import jax
import jax.numpy as jnp
from jax import lax
from jax.experimental import pallas as pl
from jax.experimental.pallas import tpu as pltpu

N_DEV = 32
N_TOK = 2048
D_IN = 512
D_OUT = 1024
N_EXP = 128
E_LOCAL = 4
CAP = 12

CW_STEPS = N_DEV // 2
CCW_STEPS = N_DEV // 2 - 1


def kernel(x, router_W, route_idx, expert_W):
    del router_W

    def body(x_ref, idx_ref, w_ref, out_ref, g_ref,
             cw_send_sem, ccw_send_sem, cw_sems, ccw_sems):
        my = lax.axis_index("i")
        left = lax.rem(my - 1 + N_DEV, N_DEV)
        right = lax.rem(my + 1, N_DEV)

        barrier_sem = pltpu.get_barrier_semaphore()
        pl.semaphore_signal(barrier_sem, inc=1, device_id=(left,),
                            device_id_type=pl.DeviceIdType.MESH)
        pl.semaphore_signal(barrier_sem, inc=1, device_id=(right,),
                            device_id_type=pl.DeviceIdType.MESH)
        pl.semaphore_wait(barrier_sem, 2)

        route = idx_ref[:, :]
        eall = lax.broadcasted_iota(jnp.int32, (1, N_EXP), 1)
        oh_all = (route == eall).astype(jnp.float32)
        row = lax.broadcasted_iota(jnp.int32, (N_TOK, N_TOK), 0)
        col = lax.broadcasted_iota(jnp.int32, (N_TOK, N_TOK), 1)
        tri = (col <= row).astype(jnp.float32)
        rank_all = jnp.dot(tri, oh_all, preferred_element_type=jnp.float32)

        base = my * E_LOCAL
        e_loc = base + lax.broadcasted_iota(jnp.int32, (1, E_LOCAL), 1)
        oh_loc = (route == e_loc).astype(jnp.float32)
        rank_loc = jnp.dot(tri, oh_loc, preferred_element_type=jnp.float32)

        xv = x_ref[:, :]
        rr = (lax.broadcasted_iota(jnp.int32, (1, CAP), 1) + 1).astype(
            jnp.float32)
        for l in range(E_LOCAL):
            sel = oh_loc[:, l:l + 1] * (rank_loc[:, l:l + 1] == rr).astype(
                jnp.float32)
            xc = lax.dot_general(
                sel, xv, dimension_numbers=(((0,), (0,)), ((), ())),
                preferred_element_type=jnp.float32)
            yc = jnp.dot(xc, w_ref[l], preferred_element_type=jnp.float32)
            g_ref[pl.ds(my, 1), l, :, :] = yc.reshape(1, CAP, D_OUT)

        for t in range(CW_STEPS):
            bs_cw = lax.rem(my - t + N_DEV, N_DEV)
            cw = pltpu.make_async_remote_copy(
                src_ref=g_ref.at[pl.ds(bs_cw, 1)],
                dst_ref=g_ref.at[pl.ds(bs_cw, 1)],
                send_sem=cw_send_sem,
                recv_sem=cw_sems.at[t],
                device_id=(right,),
                device_id_type=pl.DeviceIdType.MESH,
            )
            cw.start()
            if t < CCW_STEPS:
                bs_ccw = lax.rem(my + t, N_DEV)
                ccw = pltpu.make_async_remote_copy(
                    src_ref=g_ref.at[pl.ds(bs_ccw, 1)],
                    dst_ref=g_ref.at[pl.ds(bs_ccw, 1)],
                    send_sem=ccw_send_sem,
                    recv_sem=ccw_sems.at[t],
                    device_id=(left,),
                    device_id_type=pl.DeviceIdType.MESH,
                )
                ccw.start()
                ccw.wait()
            cw.wait()

        p_blocks = []
        g_blocks = []
        for r in range(CAP):
            p_blocks.append(
                oh_all * (rank_all == float(r + 1)).astype(jnp.float32))
            g_blocks.append(g_ref[:, :, r, :].reshape(N_EXP, D_OUT))
        p_all = jnp.concatenate(p_blocks, axis=1)
        g_all = jnp.concatenate(g_blocks, axis=0)
        out_ref[:, :] = jnp.dot(p_all, g_all,
                                preferred_element_type=jnp.float32)

    return pl.pallas_call(
        body,
        out_shape=jax.ShapeDtypeStruct((N_TOK, D_OUT), jnp.float32),
        in_specs=[
            pl.BlockSpec(memory_space=pltpu.VMEM),
            pl.BlockSpec(memory_space=pltpu.VMEM),
            pl.BlockSpec(memory_space=pltpu.VMEM),
        ],
        out_specs=pl.BlockSpec(memory_space=pltpu.VMEM),
        scratch_shapes=[
            pltpu.VMEM((N_DEV, E_LOCAL, CAP, D_OUT), jnp.float32),
            pltpu.SemaphoreType.DMA,
            pltpu.SemaphoreType.DMA,
            pltpu.SemaphoreType.DMA((CW_STEPS,)),
            pltpu.SemaphoreType.DMA((CCW_STEPS,)),
        ],
        compiler_params=pltpu.CompilerParams(
            collective_id=0,
            vmem_limit_bytes=100 * 1024 * 1024,
        ),
    )(x, route_idx, expert_W)


# device time: 128528 ns/iter; 2.5381x vs baseline; 1.1659x over previous
import jax
import jax.numpy as jnp
from jax import lax
from jax.experimental import pallas as pl
from jax.experimental.pallas import tpu as pltpu

N_DEV = 32
N_TOK = 2048
D_IN = 512
D_OUT = 1024
N_EXP = 128
E_LOCAL = 4
CAP = 12

CW_STEPS = N_DEV // 2
CCW_STEPS = N_DEV // 2 - 1


def kernel(x, router_W, route_idx, expert_W):
    del router_W

    def body(x_ref, idx_ref, w_ref, out_ref, g_ref,
             cw_send_sem, ccw_send_sem, cw_sems, ccw_sems):
        my = lax.axis_index("i")
        left = lax.rem(my - 1 + N_DEV, N_DEV)
        right = lax.rem(my + 1, N_DEV)

        barrier_sem = pltpu.get_barrier_semaphore()
        pl.semaphore_signal(barrier_sem, inc=1, device_id=(left,),
                            device_id_type=pl.DeviceIdType.MESH)
        pl.semaphore_signal(barrier_sem, inc=1, device_id=(right,),
                            device_id_type=pl.DeviceIdType.MESH)
        pl.semaphore_wait(barrier_sem, 2)

        route = idx_ref[:, :]
        eall = lax.broadcasted_iota(jnp.int32, (1, N_EXP), 1)
        oh_all = (route == eall).astype(jnp.bfloat16)
        row = lax.broadcasted_iota(jnp.int32, (N_TOK, N_TOK), 0)
        col = lax.broadcasted_iota(jnp.int32, (N_TOK, N_TOK), 1)
        tri = (col <= row).astype(jnp.bfloat16)
        rank_all = jnp.dot(tri, oh_all, preferred_element_type=jnp.float32)

        base = my * E_LOCAL
        e_loc = base + lax.broadcasted_iota(jnp.int32, (1, E_LOCAL), 1)
        oh_loc = (route == e_loc).astype(jnp.bfloat16)
        rank_loc = jnp.dot(tri, oh_loc, preferred_element_type=jnp.float32)

        xv = x_ref[:, :].astype(jnp.bfloat16)
        rr = (lax.broadcasted_iota(jnp.int32, (1, CAP), 1) + 1).astype(
            jnp.float32)
        for l in range(E_LOCAL):
            sel = (oh_loc[:, l:l + 1].astype(jnp.float32)
                   * (rank_loc[:, l:l + 1] == rr).astype(jnp.float32)
                   ).astype(jnp.bfloat16)
            xc = lax.dot_general(
                sel, xv, dimension_numbers=(((0,), (0,)), ((), ())),
                preferred_element_type=jnp.float32)
            yc = jnp.dot(xc.astype(jnp.bfloat16),
                         w_ref[l].astype(jnp.bfloat16),
                         preferred_element_type=jnp.float32)
            g_ref[pl.ds(my, 1), l, :, :] = yc.astype(jnp.bfloat16).reshape(
                1, CAP, D_OUT)

        for t in range(CW_STEPS):
            bs_cw = lax.rem(my - t + N_DEV, N_DEV)
            cw = pltpu.make_async_remote_copy(
                src_ref=g_ref.at[pl.ds(bs_cw, 1)],
                dst_ref=g_ref.at[pl.ds(bs_cw, 1)],
                send_sem=cw_send_sem,
                recv_sem=cw_sems.at[t],
                device_id=(right,),
                device_id_type=pl.DeviceIdType.MESH,
            )
            cw.start()
            if t < CCW_STEPS:
                bs_ccw = lax.rem(my + t, N_DEV)
                ccw = pltpu.make_async_remote_copy(
                    src_ref=g_ref.at[pl.ds(bs_ccw, 1)],
                    dst_ref=g_ref.at[pl.ds(bs_ccw, 1)],
                    send_sem=ccw_send_sem,
                    recv_sem=ccw_sems.at[t],
                    device_id=(left,),
                    device_id_type=pl.DeviceIdType.MESH,
                )
                ccw.start()
                ccw.wait()
            cw.wait()

        p_blocks = []
        g_blocks = []
        for r in range(CAP):
            p_blocks.append(
                oh_all * (rank_all == float(r + 1)).astype(jnp.bfloat16))
            g_blocks.append(g_ref[:, :, r, :].reshape(N_EXP, D_OUT))
        p_all = jnp.concatenate(p_blocks, axis=1)
        g_all = jnp.concatenate(g_blocks, axis=0)
        out_ref[:, :] = jnp.dot(p_all, g_all,
                                preferred_element_type=jnp.float32)

    return pl.pallas_call(
        body,
        out_shape=jax.ShapeDtypeStruct((N_TOK, D_OUT), jnp.float32),
        in_specs=[
            pl.BlockSpec(memory_space=pltpu.VMEM),
            pl.BlockSpec(memory_space=pltpu.VMEM),
            pl.BlockSpec(memory_space=pltpu.VMEM),
        ],
        out_specs=pl.BlockSpec(memory_space=pltpu.VMEM),
        scratch_shapes=[
            pltpu.VMEM((N_DEV, E_LOCAL, CAP, D_OUT), jnp.bfloat16),
            pltpu.SemaphoreType.DMA,
            pltpu.SemaphoreType.DMA,
            pltpu.SemaphoreType.DMA((CW_STEPS,)),
            pltpu.SemaphoreType.DMA((CCW_STEPS,)),
        ],
        compiler_params=pltpu.CompilerParams(
            collective_id=0,
            vmem_limit_bytes=100 * 1024 * 1024,
        ),
    )(x, route_idx, expert_W)


# device time: 84183 ns/iter; 3.8751x vs baseline; 1.5268x over previous
import jax
import jax.numpy as jnp
from jax import lax
from jax.experimental import pallas as pl
from jax.experimental.pallas import tpu as pltpu

N_DEV = 32
PLANE = 8
N_PLANES = N_DEV // PLANE
N_TOK = 2048
D_IN = 512
D_OUT = 1024
N_EXP = 128
E_LOCAL = 4
CAP = 12


def _cumsum_rows(a):
    n = a.shape[0]
    k = 1
    while k < n:
        shifted = jnp.concatenate(
            [jnp.zeros((k, a.shape[1]), a.dtype), a[:-k, :]], axis=0)
        a = a + shifted
        k *= 2
    return a


def kernel(x, router_W, route_idx, expert_W):
    del router_W

    def body(x_ref, idx_ref, w_ref, out_ref, g_ref,
             a_send_sems, a_recv_sems, b_send_sems, b_recv_sems,
             c_send_sems, c_recv_sems):
        my = lax.axis_index("i")
        plane = lax.div(my, PLANE)
        q = lax.rem(my, PLANE)

        barrier_sem = pltpu.get_barrier_semaphore()
        for d in range(1, N_DEV):
            tgt = lax.rem(my + d, N_DEV)
            pl.semaphore_signal(barrier_sem, inc=1, device_id=(tgt,),
                                device_id_type=pl.DeviceIdType.MESH)
        pl.semaphore_wait(barrier_sem, N_DEV - 1)

        route = idx_ref[:, :]

        base = my * E_LOCAL
        e_loc = base + lax.broadcasted_iota(jnp.int32, (1, E_LOCAL), 1)
        oh_loc = (route == e_loc).astype(jnp.float32)
        rank_loc = _cumsum_rows(oh_loc)

        xv = x_ref[:, :].astype(jnp.bfloat16)
        rr = (lax.broadcasted_iota(jnp.int32, (1, CAP), 1) + 1).astype(
            jnp.float32)
        for l in range(E_LOCAL):
            sel = (oh_loc[:, l:l + 1]
                   * (rank_loc[:, l:l + 1] == rr).astype(jnp.float32)
                   ).astype(jnp.bfloat16)
            xc = lax.dot_general(
                sel, xv, dimension_numbers=(((0,), (0,)), ((), ())),
                preferred_element_type=jnp.float32)
            yc = jnp.dot(xc.astype(jnp.bfloat16),
                         w_ref[l].astype(jnp.bfloat16),
                         preferred_element_type=jnp.float32)
            g_ref[pl.ds(my, 1), l, :, :] = yc.astype(jnp.bfloat16).reshape(
                1, CAP, D_OUT)

        pa = []
        for d in range(1, PLANE):
            tgt = plane * PLANE + lax.rem(q + d, PLANE)
            r = pltpu.make_async_remote_copy(
                src_ref=g_ref.at[pl.ds(my, 1)],
                dst_ref=g_ref.at[pl.ds(my, 1)],
                send_sem=a_send_sems.at[d - 1],
                recv_sem=a_recv_sems.at[d - 1],
                device_id=(tgt,),
                device_id_type=pl.DeviceIdType.MESH,
            )
            r.start()
            pa.append(r)

        pb = []
        for d in range(1, N_PLANES):
            tgt = lax.rem(my + PLANE * d, N_DEV)
            r = pltpu.make_async_remote_copy(
                src_ref=g_ref.at[pl.ds(my, 1)],
                dst_ref=g_ref.at[pl.ds(my, 1)],
                send_sem=b_send_sems.at[d - 1],
                recv_sem=b_recv_sems.at[d - 1],
                device_id=(tgt,),
                device_id_type=pl.DeviceIdType.MESH,
            )
            r.start()
            pb.append(r)

        eall = lax.broadcasted_iota(jnp.int32, (1, N_EXP), 1)
        oh_all = (route == eall).astype(jnp.bfloat16)
        m_all = _cumsum_rows(oh_all) * oh_all

        pc = []
        for d in range(1, N_PLANES):
            pb[d - 1].wait()
            src_plane = lax.rem(plane - d + N_PLANES, N_PLANES)
            slot = src_plane * PLANE + q
            for dd in range(1, PLANE):
                tgt = plane * PLANE + lax.rem(q + dd, PLANE)
                s = (dd - 1) * (N_PLANES - 1) + (d - 1)
                r = pltpu.make_async_remote_copy(
                    src_ref=g_ref.at[pl.ds(slot, 1)],
                    dst_ref=g_ref.at[pl.ds(slot, 1)],
                    send_sem=c_send_sems.at[s],
                    recv_sem=c_recv_sems.at[s],
                    device_id=(tgt,),
                    device_id_type=pl.DeviceIdType.MESH,
                )
                r.start()
                pc.append(r)

        p_blocks = []
        for r in range(CAP):
            p_blocks.append(
                (m_all == jnp.bfloat16(r + 1)).astype(jnp.bfloat16))
        p_all = jnp.concatenate(p_blocks, axis=1)

        for r in pa:
            r.wait()
        for r in pc:
            r.wait()

        g_blocks = []
        for r in range(CAP):
            g_blocks.append(g_ref[:, :, r, :].reshape(N_EXP, D_OUT))
        g_all = jnp.concatenate(g_blocks, axis=0)
        out_ref[:, :] = jnp.dot(p_all, g_all,
                                preferred_element_type=jnp.float32)

    return pl.pallas_call(
        body,
        out_shape=jax.ShapeDtypeStruct((N_TOK, D_OUT), jnp.float32),
        in_specs=[
            pl.BlockSpec(memory_space=pltpu.VMEM),
            pl.BlockSpec(memory_space=pltpu.VMEM),
            pl.BlockSpec(memory_space=pltpu.VMEM),
        ],
        out_specs=pl.BlockSpec(memory_space=pltpu.VMEM),
        scratch_shapes=[
            pltpu.VMEM((N_DEV, E_LOCAL, CAP, D_OUT), jnp.bfloat16),
            pltpu.SemaphoreType.DMA((PLANE - 1,)),
            pltpu.SemaphoreType.DMA((PLANE - 1,)),
            pltpu.SemaphoreType.DMA((N_PLANES - 1,)),
            pltpu.SemaphoreType.DMA((N_PLANES - 1,)),
            pltpu.SemaphoreType.DMA(((PLANE - 1) * (N_PLANES - 1),)),
            pltpu.SemaphoreType.DMA(((PLANE - 1) * (N_PLANES - 1),)),
        ],
        compiler_params=pltpu.CompilerParams(
            collective_id=0,
            vmem_limit_bytes=100 * 1024 * 1024,
        ),
    )(x, route_idx, expert_W)
